# baseline (speedup 1.0000x reference)
import functools

import jax
import jax.numpy as jnp
from jax.experimental import pallas as pl
from jax.experimental.pallas import tpu as pltpu

D = 1024
FF = 2816
E = 64
TOPK = 8
R = 16
SCALING = 2.0

FT = 256
NFT = FF // FT
T = 128
NT = 192
P = NT * T


def _k1_body(x_ref, wg_ref, wu_ref, wd_ref, wr_ref,
             gu_ref, out_ref, logits_ref):
    f = pl.program_id(0)
    x = x_ref[...]
    g = jax.lax.dot_general(x, wg_ref[...], (((1,), (1,)), ((), ())),
                            preferred_element_type=jnp.float32)
    u = jax.lax.dot_general(x, wu_ref[...], (((1,), (1,)), ((), ())),
                            preferred_element_type=jnp.float32)
    gu_ref[:, 0, :] = g.astype(jnp.bfloat16)
    gu_ref[:, 1, :] = u.astype(jnp.bfloat16)
    h = ((g / (1.0 + jnp.exp(-g))) * u).astype(jnp.bfloat16)
    part = jax.lax.dot_general(h, wd_ref[...], (((1,), (1,)), ((), ())),
                               preferred_element_type=jnp.float32)

    @pl.when(f == 0)
    def _():
        out_ref[...] = part
        logits_ref[...] = jax.lax.dot_general(
            x, wr_ref[...], (((1,), (1,)), ((), ())),
            preferred_element_type=jnp.float32)

    @pl.when(f != 0)
    def _():
        out_ref[...] += part


def _base_mlp(xb, Wgb, Wub, Wdb, Wrb):
    S = xb.shape[0]
    return pl.pallas_call(
        _k1_body,
        grid=(NFT,),
        in_specs=[
            pl.BlockSpec((S, D), lambda f: (0, 0)),
            pl.BlockSpec((FT, D), lambda f: (f, 0)),
            pl.BlockSpec((FT, D), lambda f: (f, 0)),
            pl.BlockSpec((D, FT), lambda f: (0, f)),
            pl.BlockSpec((E, D), lambda f: (0, 0)),
        ],
        out_specs=[
            pl.BlockSpec((S, 2, FT), lambda f: (0, 0, f)),
            pl.BlockSpec((S, D), lambda f: (0, 0)),
            pl.BlockSpec((S, E), lambda f: (0, 0)),
        ],
        out_shape=[
            jax.ShapeDtypeStruct((S, 2, FF), jnp.bfloat16),
            jax.ShapeDtypeStruct((S, D), jnp.float32),
            jax.ShapeDtypeStruct((S, E), jnp.float32),
        ],
    )(xb, Wgb, Wub, Wdb, Wrb)


def _k3_body(te_ref, gu_ref, xs_ref, w_ref,
             ag_ref, bgt_ref, au_ref, but_ref, ad_ref, bdt_ref,
             delta_ref):
    xs = xs_ref[...]
    xag = jax.lax.dot_general(xs, ag_ref[...], (((1,), (1,)), ((), ())),
                              preferred_element_type=jnp.float32)
    gd = jax.lax.dot_general(xag.astype(jnp.bfloat16), bgt_ref[...],
                             (((1,), (0,)), ((), ())),
                             preferred_element_type=jnp.float32)
    xau = jax.lax.dot_general(xs, au_ref[...], (((1,), (1,)), ((), ())),
                              preferred_element_type=jnp.float32)
    ud = jax.lax.dot_general(xau.astype(jnp.bfloat16), but_ref[...],
                             (((1,), (0,)), ((), ())),
                             preferred_element_type=jnp.float32)
    gate = gu_ref[:, 0, :].astype(jnp.float32) + SCALING * gd
    up = gu_ref[:, 1, :].astype(jnp.float32) + SCALING * ud
    hidden = ((gate / (1.0 + jnp.exp(-gate))) * up).astype(jnp.bfloat16)
    had = jax.lax.dot_general(hidden, ad_ref[...], (((1,), (1,)), ((), ())),
                              preferred_element_type=jnp.float32)
    had = (had * (SCALING * w_ref[...])).astype(jnp.bfloat16)
    delta_ref[...] = jax.lax.dot_general(
        had, bdt_ref[...], (((1,), (0,)), ((), ())),
        preferred_element_type=jnp.float32)


def _expert_deltas(tile_expert, gus, xs_s, w_col,
                   Agb, BgTb, Aub, BuTb, Adb, BdTb):
    grid_spec = pltpu.PrefetchScalarGridSpec(
        num_scalar_prefetch=1,
        grid=(NT,),
        in_specs=[
            pl.BlockSpec((T, 2, FF), lambda j, te: (j, 0, 0)),
            pl.BlockSpec((T, D), lambda j, te: (j, 0)),
            pl.BlockSpec((T, 1), lambda j, te: (j, 0)),
            pl.BlockSpec((None, R, D), lambda j, te: (te[j], 0, 0)),
            pl.BlockSpec((None, R, FF), lambda j, te: (te[j], 0, 0)),
            pl.BlockSpec((None, R, D), lambda j, te: (te[j], 0, 0)),
            pl.BlockSpec((None, R, FF), lambda j, te: (te[j], 0, 0)),
            pl.BlockSpec((None, R, FF), lambda j, te: (te[j], 0, 0)),
            pl.BlockSpec((None, R, D), lambda j, te: (te[j], 0, 0)),
        ],
        out_specs=pl.BlockSpec((T, D), lambda j, te: (j, 0)),
    )
    return pl.pallas_call(
        _k3_body,
        grid_spec=grid_spec,
        out_shape=jax.ShapeDtypeStruct((P, D), jnp.float32),
    )(tile_expert, gus, xs_s, w_col,
      Agb, BgTb, Aub, BuTb, Adb, BdTb)


def _dispatch(sel, rw):
    S = sel.shape[0]
    onehot = (sel[:, :, None] == jnp.arange(E, dtype=sel.dtype)[None, None, :])
    onehot = onehot.any(axis=1).astype(jnp.int32)
    cumincl = jnp.cumsum(onehot, axis=0)
    counts = cumincl[-1]
    cumexcl = cumincl - onehot
    padded = ((counts + T - 1) // T) * T
    pend = jnp.cumsum(padded)
    poff = (pend - padded).astype(jnp.int32)
    rank = jnp.take_along_axis(cumexcl, sel, axis=1)
    ppos = poff[sel] + rank.astype(jnp.int32)
    tile_expert = jnp.searchsorted(
        pend, jnp.arange(NT, dtype=jnp.int32) * T, side='right')
    tile_expert = jnp.minimum(tile_expert, E - 1).astype(jnp.int32)
    ppos_flat = ppos.reshape(-1)
    slot_token = jnp.zeros((P,), jnp.int32).at[ppos_flat].set(
        jnp.arange(S * TOPK, dtype=jnp.int32) // TOPK)
    slot_w = jnp.zeros((P,), jnp.float32).at[ppos_flat].set(rw.reshape(-1))
    return tile_expert, slot_token, slot_w, ppos_flat


def kernel(x, Wg, Wu, Wd, Wr, Ag, Bg, Au, Bu, Ad, Bd):
    b, s, d = x.shape
    xf = x.reshape(-1, d)
    xb = xf.astype(jnp.bfloat16)

    gus, base_out, logits = _base_mlp(
        xb, Wg.astype(jnp.bfloat16), Wu.astype(jnp.bfloat16),
        Wd.astype(jnp.bfloat16), Wr.astype(jnp.bfloat16))

    probs = jax.nn.softmax(logits, axis=-1)
    rw, sel = jax.lax.top_k(probs, TOPK)

    tile_expert, slot_token, slot_w, ppos_flat = _dispatch(sel, rw)

    gus_s = gus[slot_token]
    xs_s = xb[slot_token]

    Agb = Ag.astype(jnp.bfloat16)
    Aub = Au.astype(jnp.bfloat16)
    Adb = Ad.astype(jnp.bfloat16)
    BgTb = jnp.swapaxes(Bg, 1, 2).astype(jnp.bfloat16)
    BuTb = jnp.swapaxes(Bu, 1, 2).astype(jnp.bfloat16)
    BdTb = jnp.swapaxes(Bd, 1, 2).astype(jnp.bfloat16)

    delta = _expert_deltas(tile_expert, gus_s, xs_s,
                           slot_w.reshape(P, 1), Agb, BgTb, Aub, BuTb,
                           Adb, BdTb)

    expert_out = delta[ppos_flat].reshape(b * s, TOPK, d).sum(axis=1)
    return (base_out + expert_out).reshape(b, s, d)


# speedup vs baseline: 1.3071x; 1.3071x over previous
import functools

import jax
import jax.numpy as jnp
from jax.experimental import pallas as pl
from jax.experimental.pallas import tpu as pltpu

D = 1024
FF = 2816
E = 64
TOPK = 8
R = 16
SCALING = 2.0

FT = 256
NFT = FF // FT
T = 128
NT = 192
P = NT * T


def _k1_body(x_ref, wg_ref, wu_ref, wd_ref, wr_ref,
             gb_ref, ub_ref, out_ref, logits_ref):
    f = pl.program_id(0)
    x = x_ref[...]
    g = jax.lax.dot_general(x, wg_ref[...], (((1,), (1,)), ((), ())),
                            preferred_element_type=jnp.float32)
    u = jax.lax.dot_general(x, wu_ref[...], (((1,), (1,)), ((), ())),
                            preferred_element_type=jnp.float32)
    gb_ref[...] = g.astype(jnp.bfloat16)
    ub_ref[...] = u.astype(jnp.bfloat16)
    h = ((g / (1.0 + jnp.exp(-g))) * u).astype(jnp.bfloat16)
    part = jax.lax.dot_general(h, wd_ref[...], (((1,), (1,)), ((), ())),
                               preferred_element_type=jnp.float32)

    @pl.when(f == 0)
    def _():
        out_ref[...] = part
        logits_ref[...] = jax.lax.dot_general(
            x, wr_ref[...], (((1,), (1,)), ((), ())),
            preferred_element_type=jnp.float32)

    @pl.when(f != 0)
    def _():
        out_ref[...] += part


def _base_mlp(xb, Wgb, Wub, Wdb, Wrb):
    S = xb.shape[0]
    return pl.pallas_call(
        _k1_body,
        grid=(NFT,),
        in_specs=[
            pl.BlockSpec((S, D), lambda f: (0, 0)),
            pl.BlockSpec((FT, D), lambda f: (f, 0)),
            pl.BlockSpec((FT, D), lambda f: (f, 0)),
            pl.BlockSpec((D, FT), lambda f: (0, f)),
            pl.BlockSpec((E, D), lambda f: (0, 0)),
        ],
        out_specs=[
            pl.BlockSpec((S, FT), lambda f: (0, f)),
            pl.BlockSpec((S, FT), lambda f: (0, f)),
            pl.BlockSpec((S, D), lambda f: (0, 0)),
            pl.BlockSpec((S, E), lambda f: (0, 0)),
        ],
        out_shape=[
            jax.ShapeDtypeStruct((S, FF), jnp.bfloat16),
            jax.ShapeDtypeStruct((S, FF), jnp.bfloat16),
            jax.ShapeDtypeStruct((S, D), jnp.float32),
            jax.ShapeDtypeStruct((S, E), jnp.float32),
        ],
    )(xb, Wgb, Wub, Wdb, Wrb)


def _k3_body(te_ref, gs_ref, us_ref, xs_ref, w_ref,
             ag_ref, bgt_ref, au_ref, but_ref, ad_ref, bdt_ref,
             delta_ref):
    xs = xs_ref[...]
    xag = jax.lax.dot_general(xs, ag_ref[...], (((1,), (1,)), ((), ())),
                              preferred_element_type=jnp.float32)
    gd = jax.lax.dot_general(xag.astype(jnp.bfloat16), bgt_ref[...],
                             (((1,), (0,)), ((), ())),
                             preferred_element_type=jnp.float32)
    xau = jax.lax.dot_general(xs, au_ref[...], (((1,), (1,)), ((), ())),
                              preferred_element_type=jnp.float32)
    ud = jax.lax.dot_general(xau.astype(jnp.bfloat16), but_ref[...],
                             (((1,), (0,)), ((), ())),
                             preferred_element_type=jnp.float32)
    gate = gs_ref[...].astype(jnp.float32) + SCALING * gd
    up = us_ref[...].astype(jnp.float32) + SCALING * ud
    hidden = ((gate / (1.0 + jnp.exp(-gate))) * up).astype(jnp.bfloat16)
    had = jax.lax.dot_general(hidden, ad_ref[...], (((1,), (1,)), ((), ())),
                              preferred_element_type=jnp.float32)
    had = (had * (SCALING * w_ref[...])).astype(jnp.bfloat16)
    delta_ref[...] = jax.lax.dot_general(
        had, bdt_ref[...], (((1,), (0,)), ((), ())),
        preferred_element_type=jnp.float32).astype(jnp.bfloat16)


def _expert_deltas(tile_expert, gs_s, us_s, xs_s, w_col,
                   Agb, BgTb, Aub, BuTb, Adb, BdTb):
    grid_spec = pltpu.PrefetchScalarGridSpec(
        num_scalar_prefetch=1,
        grid=(NT,),
        in_specs=[
            pl.BlockSpec((T, FF), lambda j, te: (j, 0)),
            pl.BlockSpec((T, FF), lambda j, te: (j, 0)),
            pl.BlockSpec((T, D), lambda j, te: (j, 0)),
            pl.BlockSpec((T, 1), lambda j, te: (j, 0)),
            pl.BlockSpec((None, R, D), lambda j, te: (te[j], 0, 0)),
            pl.BlockSpec((None, R, FF), lambda j, te: (te[j], 0, 0)),
            pl.BlockSpec((None, R, D), lambda j, te: (te[j], 0, 0)),
            pl.BlockSpec((None, R, FF), lambda j, te: (te[j], 0, 0)),
            pl.BlockSpec((None, R, FF), lambda j, te: (te[j], 0, 0)),
            pl.BlockSpec((None, R, D), lambda j, te: (te[j], 0, 0)),
        ],
        out_specs=pl.BlockSpec((T, D), lambda j, te: (j, 0)),
    )
    return pl.pallas_call(
        _k3_body,
        grid_spec=grid_spec,
        out_shape=jax.ShapeDtypeStruct((P, D), jnp.bfloat16),
    )(tile_expert, gs_s, us_s, xs_s, w_col,
      Agb, BgTb, Aub, BuTb, Adb, BdTb)


def _dispatch(sel, rw):
    S = sel.shape[0]
    onehot = (sel[:, :, None] == jnp.arange(E, dtype=sel.dtype)[None, None, :])
    onehot = onehot.any(axis=1).astype(jnp.int32)
    cumincl = jnp.cumsum(onehot, axis=0)
    counts = cumincl[-1]
    cumexcl = cumincl - onehot
    padded = ((counts + T - 1) // T) * T
    pend = jnp.cumsum(padded)
    poff = (pend - padded).astype(jnp.int32)
    rank = jnp.take_along_axis(cumexcl, sel, axis=1)
    ppos = poff[sel] + rank.astype(jnp.int32)
    tile_expert = jnp.searchsorted(
        pend, jnp.arange(NT, dtype=jnp.int32) * T, side='right')
    tile_expert = jnp.minimum(tile_expert, E - 1).astype(jnp.int32)
    ppos_flat = ppos.reshape(-1)
    slot_token = jnp.zeros((P,), jnp.int32).at[ppos_flat].set(
        jnp.arange(S * TOPK, dtype=jnp.int32) // TOPK)
    slot_w = jnp.zeros((P,), jnp.float32).at[ppos_flat].set(rw.reshape(-1))
    return tile_expert, slot_token, slot_w, ppos_flat


def kernel(x, Wg, Wu, Wd, Wr, Ag, Bg, Au, Bu, Ad, Bd):
    b, s, d = x.shape
    xf = x.reshape(-1, d)
    xb = xf.astype(jnp.bfloat16)

    gate_b, up_b, base_out, logits = _base_mlp(
        xb, Wg.astype(jnp.bfloat16), Wu.astype(jnp.bfloat16),
        Wd.astype(jnp.bfloat16), Wr.astype(jnp.bfloat16))

    probs = jax.nn.softmax(logits, axis=-1)
    rw, sel = jax.lax.top_k(probs, TOPK)

    tile_expert, slot_token, slot_w, ppos_flat = _dispatch(sel, rw)

    gs_s = gate_b[slot_token]
    us_s = up_b[slot_token]
    xs_s = xb[slot_token]

    Agb = Ag.astype(jnp.bfloat16)
    Aub = Au.astype(jnp.bfloat16)
    Adb = Ad.astype(jnp.bfloat16)
    BgTb = jnp.swapaxes(Bg, 1, 2).astype(jnp.bfloat16)
    BuTb = jnp.swapaxes(Bu, 1, 2).astype(jnp.bfloat16)
    BdTb = jnp.swapaxes(Bd, 1, 2).astype(jnp.bfloat16)

    delta = _expert_deltas(tile_expert, gs_s, us_s, xs_s,
                           slot_w.reshape(P, 1), Agb, BgTb, Aub, BuTb,
                           Adb, BdTb)

    expert_out = delta[ppos_flat].astype(jnp.float32).reshape(
        b * s, TOPK, d).sum(axis=1)
    return (base_out + expert_out).reshape(b, s, d)


# speedup vs baseline: 1.3938x; 1.0663x over previous
import functools

import numpy as np
import jax
import jax.numpy as jnp
from jax import lax
from jax.experimental import pallas as pl
from jax.experimental.pallas import tpu as pltpu
from jax.experimental.pallas import tpu_sc as plsc

D = 1024
FF = 2816
E = 64
TOPK = 8
R = 16
SCALING = 2.0

FT = 256
NFT = FF // FT
HFT = FT // 2
FH = FF // 2
DH = D // 2
T = 128
NT = 192
P = NT * T
S_TOK = 2048
NPAIR = S_TOK * TOPK

NC, NS = 2, 16
NW = NC * NS

_PERM_FF = np.concatenate([
    (np.arange(NFT)[:, None] * FT + np.arange(HFT)[None, :]).reshape(-1),
    (np.arange(NFT)[:, None] * FT + HFT + np.arange(HFT)[None, :]).reshape(-1),
])


def _pack(lo, hi):
    lo_u = lax.bitcast_convert_type(
        lo.astype(jnp.bfloat16).astype(jnp.float32), jnp.uint32)
    hi_u = lax.bitcast_convert_type(
        hi.astype(jnp.bfloat16).astype(jnp.float32), jnp.uint32)
    packed = (hi_u & jnp.uint32(0xFFFF0000)) | (lo_u >> 16)
    return lax.bitcast_convert_type(packed, jnp.int32)


def _unpack(packed_i32):
    u = lax.bitcast_convert_type(packed_i32, jnp.uint32)
    lo = lax.bitcast_convert_type(u << 16, jnp.float32)
    hi = lax.bitcast_convert_type(u & jnp.uint32(0xFFFF0000), jnp.float32)
    return jnp.concatenate([lo, hi], axis=1)


def _k1_body(x_ref, wg_ref, wu_ref, wd_ref, wr_ref,
             gp_ref, up_ref, xp_ref, out_ref, logits_ref):
    f = pl.program_id(0)
    x = x_ref[...]
    xb = x.astype(jnp.bfloat16)
    g = jax.lax.dot_general(xb, wg_ref[...], (((1,), (1,)), ((), ())),
                            preferred_element_type=jnp.float32)
    u = jax.lax.dot_general(xb, wu_ref[...], (((1,), (1,)), ((), ())),
                            preferred_element_type=jnp.float32)
    gp_ref[...] = _pack(g[:, :HFT], g[:, HFT:])
    up_ref[...] = _pack(u[:, :HFT], u[:, HFT:])
    h = ((g / (1.0 + jnp.exp(-g))) * u).astype(jnp.bfloat16)
    part = jax.lax.dot_general(h, wd_ref[...], (((1,), (1,)), ((), ())),
                               preferred_element_type=jnp.float32)

    @pl.when(f == 0)
    def _():
        out_ref[...] = part
        logits_ref[...] = jax.lax.dot_general(
            xb, wr_ref[...], (((1,), (1,)), ((), ())),
            preferred_element_type=jnp.float32)
        xp_ref[...] = _pack(x[:, :DH], x[:, DH:])

    @pl.when(f != 0)
    def _():
        out_ref[...] += part


def _base_mlp(xf, Wgb, Wub, Wdb, Wrb):
    S = xf.shape[0]
    return pl.pallas_call(
        _k1_body,
        grid=(NFT,),
        in_specs=[
            pl.BlockSpec((S, D), lambda f: (0, 0)),
            pl.BlockSpec((FT, D), lambda f: (f, 0)),
            pl.BlockSpec((FT, D), lambda f: (f, 0)),
            pl.BlockSpec((D, FT), lambda f: (0, f)),
            pl.BlockSpec((E, D), lambda f: (0, 0)),
        ],
        out_specs=[
            pl.BlockSpec((S, HFT), lambda f: (0, f)),
            pl.BlockSpec((S, HFT), lambda f: (0, f)),
            pl.BlockSpec((S, DH), lambda f: (0, 0)),
            pl.BlockSpec((S, D), lambda f: (0, 0)),
            pl.BlockSpec((S, E), lambda f: (0, 0)),
        ],
        out_shape=[
            jax.ShapeDtypeStruct((S, FH), jnp.int32),
            jax.ShapeDtypeStruct((S, FH), jnp.int32),
            jax.ShapeDtypeStruct((S, DH), jnp.int32),
            jax.ShapeDtypeStruct((S, D), jnp.float32),
            jax.ShapeDtypeStruct((S, E), jnp.float32),
        ],
    )(xf, Wgb, Wub, Wdb, Wrb)


def _dispatch(sel, rw):
    S = sel.shape[0]
    onehot = (sel[:, :, None] == jnp.arange(E, dtype=sel.dtype)[None, None, :])
    onehot = onehot.any(axis=1).astype(jnp.int32)
    cumincl = jnp.cumsum(onehot, axis=0)
    counts = cumincl[-1]
    cumexcl = cumincl - onehot
    padded = ((counts + T - 1) // T) * T
    pend = jnp.cumsum(padded)
    poff = (pend - padded).astype(jnp.int32)
    rank = jnp.take_along_axis(cumexcl, sel, axis=1)
    ppos = poff[sel] + rank.astype(jnp.int32)
    tile_expert = jnp.searchsorted(
        pend, jnp.arange(NT, dtype=jnp.int32) * T, side='right')
    tile_expert = jnp.minimum(tile_expert, E - 1).astype(jnp.int32)
    ppos_flat = ppos.reshape(-1)
    slot_token = jnp.zeros((P,), jnp.int32).at[ppos_flat].set(
        jnp.arange(NPAIR, dtype=jnp.int32) // TOPK)
    slot_w = jnp.zeros((P,), jnp.float32).at[ppos_flat].set(rw.reshape(-1))
    return tile_expert, slot_token, slot_w, ppos_flat


_K2_CH = 32
_K2_PER_W = P // NW


def _k2_body(gp_hbm, up_hbm, xp_hbm, tok_hbm,
             gs_hbm, us_hbm, xs_hbm,
             idx_v, g_v, u_v, x_v, sg, su, sx):
    wid = lax.axis_index("s") * NC + lax.axis_index("c")
    base = wid * _K2_PER_W

    def chunk(i, _):
        off = base + i * _K2_CH
        pltpu.sync_copy(tok_hbm.at[pl.ds(off, _K2_CH)], idx_v)
        cg = pltpu.async_copy(gp_hbm.at[idx_v], g_v, sg)
        cu = pltpu.async_copy(up_hbm.at[idx_v], u_v, su)
        cx = pltpu.async_copy(xp_hbm.at[idx_v], x_v, sx)
        cg.wait()
        cu.wait()
        cx.wait()
        pltpu.sync_copy(g_v, gs_hbm.at[pl.ds(off, _K2_CH)])
        pltpu.sync_copy(u_v, us_hbm.at[pl.ds(off, _K2_CH)])
        pltpu.sync_copy(x_v, xs_hbm.at[pl.ds(off, _K2_CH)])
        return ()

    lax.fori_loop(0, _K2_PER_W // _K2_CH, chunk, ())


def _stage_gather(gate_p, up_p, x_p, slot_token):
    mesh = plsc.VectorSubcoreMesh(core_axis_name="c", subcore_axis_name="s")
    return pl.kernel(
        _k2_body,
        out_type=(
            jax.ShapeDtypeStruct((P, FH), jnp.int32),
            jax.ShapeDtypeStruct((P, FH), jnp.int32),
            jax.ShapeDtypeStruct((P, DH), jnp.int32),
        ),
        mesh=mesh,
        scratch_types=[
            pltpu.VMEM((_K2_CH,), jnp.int32),
            pltpu.VMEM((_K2_CH, FH), jnp.int32),
            pltpu.VMEM((_K2_CH, FH), jnp.int32),
            pltpu.VMEM((_K2_CH, DH), jnp.int32),
            pltpu.SemaphoreType.DMA,
            pltpu.SemaphoreType.DMA,
            pltpu.SemaphoreType.DMA,
        ],
    )(gate_p, up_p, x_p, slot_token)


def _k3_body(te_ref, gs_ref, us_ref, xs_ref, w_ref,
             ag_ref, bgt_ref, au_ref, but_ref, ad_ref, bdt_ref,
             delta_ref):
    xs = _unpack(xs_ref[...]).astype(jnp.bfloat16)
    xag = jax.lax.dot_general(xs, ag_ref[...], (((1,), (1,)), ((), ())),
                              preferred_element_type=jnp.float32)
    gd = jax.lax.dot_general(xag.astype(jnp.bfloat16), bgt_ref[...],
                             (((1,), (0,)), ((), ())),
                             preferred_element_type=jnp.float32)
    xau = jax.lax.dot_general(xs, au_ref[...], (((1,), (1,)), ((), ())),
                              preferred_element_type=jnp.float32)
    ud = jax.lax.dot_general(xau.astype(jnp.bfloat16), but_ref[...],
                             (((1,), (0,)), ((), ())),
                             preferred_element_type=jnp.float32)
    gate = _unpack(gs_ref[...]) + SCALING * gd
    up = _unpack(us_ref[...]) + SCALING * ud
    hidden = ((gate / (1.0 + jnp.exp(-gate))) * up).astype(jnp.bfloat16)
    had = jax.lax.dot_general(hidden, ad_ref[...], (((1,), (1,)), ((), ())),
                              preferred_element_type=jnp.float32)
    had = (had * (SCALING * w_ref[...])).astype(jnp.bfloat16)
    delta_ref[...] = jax.lax.dot_general(
        had, bdt_ref[...], (((1,), (0,)), ((), ())),
        preferred_element_type=jnp.float32)


def _expert_deltas(tile_expert, gs_s, us_s, xs_s, w_col,
                   Agb, BgTb, Aub, BuTb, Adb, BdTb):
    grid_spec = pltpu.PrefetchScalarGridSpec(
        num_scalar_prefetch=1,
        grid=(NT,),
        in_specs=[
            pl.BlockSpec((T, FH), lambda j, te: (j, 0)),
            pl.BlockSpec((T, FH), lambda j, te: (j, 0)),
            pl.BlockSpec((T, DH), lambda j, te: (j, 0)),
            pl.BlockSpec((T, 1), lambda j, te: (j, 0)),
            pl.BlockSpec((None, R, D), lambda j, te: (te[j], 0, 0)),
            pl.BlockSpec((None, R, FF), lambda j, te: (te[j], 0, 0)),
            pl.BlockSpec((None, R, D), lambda j, te: (te[j], 0, 0)),
            pl.BlockSpec((None, R, FF), lambda j, te: (te[j], 0, 0)),
            pl.BlockSpec((None, R, FF), lambda j, te: (te[j], 0, 0)),
            pl.BlockSpec((None, R, D), lambda j, te: (te[j], 0, 0)),
        ],
        out_specs=pl.BlockSpec((T, D), lambda j, te: (j, 0)),
    )
    return pl.pallas_call(
        _k3_body,
        grid_spec=grid_spec,
        out_shape=jax.ShapeDtypeStruct((P, D), jnp.float32),
    )(tile_expert, gs_s, us_s, xs_s, w_col,
      Agb, BgTb, Aub, BuTb, Adb, BdTb)


_K4_CH = 64
_K4_PER_W = NPAIR // NW


def _k4_body(delta_hbm, pos_hbm, dt_hbm, idx_v, rows_v, sem):
    wid = lax.axis_index("s") * NC + lax.axis_index("c")
    base = wid * _K4_PER_W

    def chunk(i, _):
        off = base + i * _K4_CH
        pltpu.sync_copy(pos_hbm.at[pl.ds(off, _K4_CH)], idx_v)
        pltpu.async_copy(delta_hbm.at[idx_v], rows_v, sem).wait()
        pltpu.sync_copy(rows_v, dt_hbm.at[pl.ds(off, _K4_CH)])
        return ()

    lax.fori_loop(0, _K4_PER_W // _K4_CH, chunk, ())


def _delta_to_token_order(delta, ppos_flat):
    mesh = plsc.VectorSubcoreMesh(core_axis_name="c", subcore_axis_name="s")
    return pl.kernel(
        _k4_body,
        out_type=jax.ShapeDtypeStruct((NPAIR, D), jnp.float32),
        mesh=mesh,
        scratch_types=[
            pltpu.VMEM((_K4_CH,), jnp.int32),
            pltpu.VMEM((_K4_CH, D), jnp.float32),
            pltpu.SemaphoreType.DMA,
        ],
    )(delta, ppos_flat)


_K5_T = 128


def _k5_body(dt_ref, base_ref, out_ref):
    d = dt_ref[...].reshape(_K5_T, TOPK, D)
    out_ref[...] = base_ref[...] + d.sum(axis=1)


def _combine(delta_tok, base_out):
    S = base_out.shape[0]
    return pl.pallas_call(
        _k5_body,
        grid=(S // _K5_T,),
        in_specs=[
            pl.BlockSpec((_K5_T * TOPK, D), lambda i: (i, 0)),
            pl.BlockSpec((_K5_T, D), lambda i: (i, 0)),
        ],
        out_specs=pl.BlockSpec((_K5_T, D), lambda i: (i, 0)),
        out_shape=jax.ShapeDtypeStruct((S, D), jnp.float32),
    )(delta_tok, base_out)


def kernel(x, Wg, Wu, Wd, Wr, Ag, Bg, Au, Bu, Ad, Bd):
    b, s, d = x.shape
    xf = x.reshape(-1, d)

    gate_p, up_p, x_p, base_out, logits = _base_mlp(
        xf, Wg.astype(jnp.bfloat16), Wu.astype(jnp.bfloat16),
        Wd.astype(jnp.bfloat16), Wr.astype(jnp.bfloat16))

    probs = jax.nn.softmax(logits, axis=-1)
    rw, sel = jax.lax.top_k(probs, TOPK)
    tile_expert, slot_token, slot_w, ppos_flat = _dispatch(sel, rw)

    gs_s, us_s, xs_s = _stage_gather(gate_p, up_p, x_p, slot_token)

    perm = jnp.asarray(_PERM_FF)
    Agb = Ag.astype(jnp.bfloat16)
    Aub = Au.astype(jnp.bfloat16)
    Adb = Ad[:, :, perm].astype(jnp.bfloat16)
    BgTb = jnp.swapaxes(Bg, 1, 2)[:, :, perm].astype(jnp.bfloat16)
    BuTb = jnp.swapaxes(Bu, 1, 2)[:, :, perm].astype(jnp.bfloat16)
    BdTb = jnp.swapaxes(Bd, 1, 2).astype(jnp.bfloat16)

    delta = _expert_deltas(tile_expert, gs_s, us_s, xs_s,
                           slot_w.reshape(P, 1), Agb, BgTb, Aub, BuTb,
                           Adb, BdTb)

    delta_tok = _delta_to_token_order(delta, ppos_flat)
    return _combine(delta_tok, base_out).reshape(b, s, d)
